# baseline (speedup 1.0000x reference)
import functools

import jax
import jax.numpy as jnp
from jax import lax
from jax.experimental import pallas as pl
from jax.experimental.pallas import tpu as pltpu
from jax.experimental.pallas import tpu_sc as plsc

_NC = 2
_NS = 16
_NW = _NC * _NS
_C1 = 104
_NBUF = 4
_L = 16


@functools.lru_cache(maxsize=None)
def _make_gather(B: int, S: int, V: int, H: int, P: int):
    D = P + H
    b_per_w = B // _NW
    n_outer = b_per_w // _NBUF
    c2 = S - _C1
    n_grp = (S + _L - 1) // _L
    s_pad = n_grp * _L
    assert B % _NW == 0 and b_per_w % _NBUF == 0
    mesh = plsc.VectorSubcoreMesh(core_axis_name="c", subcore_axis_name="s")

    @functools.partial(
        pl.kernel,
        out_type=jax.ShapeDtypeStruct((B, S, D), jnp.float32),
        mesh=mesh,
        scratch_types=(
            [pltpu.VMEM((b_per_w * S,), jnp.int32)]
            + [pltpu.VMEM((s_pad, H), jnp.float32) for _ in range(_NBUF)]
            + [pltpu.VMEM((s_pad, P), jnp.float32) for _ in range(_NBUF)]
            + [pltpu.VMEM((s_pad,), jnp.int32) for _ in range(_NBUF)]
            + [pltpu.SemaphoreType.DMA for _ in range(4 * _NBUF)]
        ),
        compiler_params=pltpu.CompilerParams(use_tc_tiling_on_sc=False,
                                             needs_layout_passes=False),
    )
    def gather(tab_hbm, bits_hbm, idx_hbm, out_hbm, idx_v, *bufs):
        trows = bufs[:_NBUF]
        pflag = bufs[_NBUF:2 * _NBUF]
        pbits = bufs[2 * _NBUF:3 * _NBUF]
        gsem_t = bufs[3 * _NBUF:4 * _NBUF]
        gsem_p = bufs[4 * _NBUF:5 * _NBUF]
        wsem = bufs[5 * _NBUF:6 * _NBUF]
        wsem_p = bufs[6 * _NBUF:7 * _NBUF]

        wid = lax.axis_index("s") * _NC + lax.axis_index("c")
        b0 = wid * b_per_w

        pltpu.sync_copy(idx_hbm.at[pl.ds(b0 * S, b_per_w * S)], idx_v)

        def start_gathers(bloc, b):
            i0 = bloc * S
            pltpu.async_copy(tab_hbm.at[idx_v.at[pl.ds(i0, _C1)]],
                             trows[b].at[pl.ds(0, _C1)], gsem_t[b])
            pltpu.async_copy(tab_hbm.at[idx_v.at[pl.ds(i0 + _C1, c2)]],
                             trows[b].at[pl.ds(_C1, c2)], gsem_t[b])
            pltpu.async_copy(bits_hbm.at[idx_v.at[pl.ds(i0, _C1)]],
                             pbits[b].at[pl.ds(0, _C1)], gsem_p[b])
            pltpu.async_copy(bits_hbm.at[idx_v.at[pl.ds(i0 + _C1, c2)]],
                             pbits[b].at[pl.ds(_C1, c2)], gsem_p[b])

        def wait_gathers(bloc, b):
            i0 = bloc * S
            pltpu.make_async_copy(tab_hbm.at[idx_v.at[pl.ds(i0, _C1)]],
                                  trows[b].at[pl.ds(0, _C1)], gsem_t[b]).wait()
            pltpu.make_async_copy(tab_hbm.at[idx_v.at[pl.ds(i0 + _C1, c2)]],
                                  trows[b].at[pl.ds(_C1, c2)], gsem_t[b]).wait()
            pltpu.make_async_copy(bits_hbm.at[idx_v.at[pl.ds(i0, _C1)]],
                                  pbits[b].at[pl.ds(0, _C1)], gsem_p[b]).wait()
            pltpu.make_async_copy(bits_hbm.at[idx_v.at[pl.ds(i0 + _C1, c2)]],
                                  pbits[b].at[pl.ds(_C1, c2)], gsem_p[b]).wait()

        lane = lax.iota(jnp.int32, _L)
        zero = lane * 0

        def unpack_flags(b):
            for g in range(n_grp):
                w = pbits[b][pl.ds(g * _L, _L)]
                rows = lane + g * _L
                for p in range(P):
                    bit = (w >> p) & 1
                    plsc.store_scatter(pflag[b], [rows, zero + p],
                                       bit.astype(jnp.float32))

        for b in range(_NBUF):
            start_gathers(b, b)

        def outer(g, carry):
            bg = g * _NBUF
            for b in range(_NBUF):
                bloc = bg + b
                bb = b0 + bloc
                wait_gathers(bloc, b)
                unpack_flags(b)
                pltpu.async_copy(
                    trows[b].at[pl.ds(0, S)],
                    out_hbm.at[bb, :, pl.ds(P, H)], wsem[b])
                pltpu.async_copy(
                    pflag[b].at[pl.ds(0, S)],
                    out_hbm.at[bb, :, pl.ds(0, P)], wsem_p[b])
            for b in range(_NBUF):
                bloc = bg + b
                bb = b0 + bloc
                pltpu.make_async_copy(
                    trows[b].at[pl.ds(0, S)],
                    out_hbm.at[bb, :, pl.ds(P, H)], wsem[b]).wait()
                pltpu.make_async_copy(
                    pflag[b].at[pl.ds(0, S)],
                    out_hbm.at[bb, :, pl.ds(0, P)], wsem_p[b]).wait()

                @pl.when(g < n_outer - 1)
                def _():
                    start_gathers(bloc + _NBUF, b)

            return carry

        lax.fori_loop(0, n_outer, outer, 0)

    return gather


def kernel(x, table, pos_vectors):
    B, S = x.shape
    V, H = table.shape
    P = pos_vectors.shape[0]
    weights = (2 ** jnp.arange(P, dtype=jnp.int32)).astype(jnp.float32)
    pos_bits = (weights @ pos_vectors).astype(jnp.int32)
    idx = x.reshape(B * S).astype(jnp.int32)
    return _make_gather(B, S, V, H, P)(table, pos_bits, idx)


# speedup vs baseline: 1.5608x; 1.5608x over previous
import functools

import jax
import jax.numpy as jnp
from jax import lax
from jax.experimental import pallas as pl
from jax.experimental.pallas import tpu as pltpu
from jax.experimental.pallas import tpu_sc as plsc

_NC = 2
_NS = 16
_NW = _NC * _NS
_C1 = 104
_NBUF = 4
_L = 16


@functools.lru_cache(maxsize=None)
def _make_gather(B: int, S: int, V: int, H: int, P: int):
    D = P + H
    b_per_w = B // _NW
    n_outer = b_per_w // _NBUF
    c2 = S - _C1
    n_grp = (S + _L - 1) // _L
    s_pad = n_grp * _L
    assert B % _NW == 0 and b_per_w % _NBUF == 0
    mesh = plsc.VectorSubcoreMesh(core_axis_name="c", subcore_axis_name="s")

    @functools.partial(
        pl.kernel,
        out_type=jax.ShapeDtypeStruct((B, S, 128), jnp.float32),
        mesh=mesh,
        scratch_types=(
            [pltpu.VMEM((b_per_w * S,), jnp.int32)]
            + [pltpu.VMEM((s_pad, H), jnp.float32) for _ in range(_NBUF)]
            + [pltpu.VMEM((s_pad, P), jnp.float32) for _ in range(_NBUF)]
            + [pltpu.VMEM((s_pad,), jnp.int32) for _ in range(_NBUF)]
            + [pltpu.SemaphoreType.DMA for _ in range(4 * _NBUF)]
        ),
        compiler_params=pltpu.CompilerParams(use_tc_tiling_on_sc=False,
                                             needs_layout_passes=False),
    )
    def gather(tab_hbm, bits_hbm, idx_hbm, out_hbm, idx_v, *bufs):
        trows = bufs[:_NBUF]
        pflag = bufs[_NBUF:2 * _NBUF]
        pbits = bufs[2 * _NBUF:3 * _NBUF]
        gsem_t = bufs[3 * _NBUF:4 * _NBUF]
        gsem_p = bufs[4 * _NBUF:5 * _NBUF]
        wsem = bufs[5 * _NBUF:6 * _NBUF]
        wsem_p = bufs[6 * _NBUF:7 * _NBUF]

        wid = lax.axis_index("s") * _NC + lax.axis_index("c")
        b0 = wid * b_per_w

        pltpu.sync_copy(idx_hbm.at[pl.ds(b0 * S, b_per_w * S)], idx_v)

        def start_gathers(bloc, b):
            i0 = bloc * S
            pltpu.async_copy(tab_hbm.at[idx_v.at[pl.ds(i0, _C1)]],
                             trows[b].at[pl.ds(0, _C1)], gsem_t[b])
            pltpu.async_copy(tab_hbm.at[idx_v.at[pl.ds(i0 + _C1, c2)]],
                             trows[b].at[pl.ds(_C1, c2)], gsem_t[b])
            pltpu.async_copy(bits_hbm.at[idx_v.at[pl.ds(i0, _C1)]],
                             pbits[b].at[pl.ds(0, _C1)], gsem_p[b])
            pltpu.async_copy(bits_hbm.at[idx_v.at[pl.ds(i0 + _C1, c2)]],
                             pbits[b].at[pl.ds(_C1, c2)], gsem_p[b])

        def wait_tab_gathers(bloc, b):
            i0 = bloc * S
            pltpu.make_async_copy(tab_hbm.at[idx_v.at[pl.ds(i0, _C1)]],
                                  trows[b].at[pl.ds(0, _C1)], gsem_t[b]).wait()
            pltpu.make_async_copy(tab_hbm.at[idx_v.at[pl.ds(i0 + _C1, c2)]],
                                  trows[b].at[pl.ds(_C1, c2)], gsem_t[b]).wait()

        def wait_bits_gathers(bloc, b):
            i0 = bloc * S
            pltpu.make_async_copy(bits_hbm.at[idx_v.at[pl.ds(i0, _C1)]],
                                  pbits[b].at[pl.ds(0, _C1)], gsem_p[b]).wait()
            pltpu.make_async_copy(bits_hbm.at[idx_v.at[pl.ds(i0 + _C1, c2)]],
                                  pbits[b].at[pl.ds(_C1, c2)], gsem_p[b]).wait()

        lane = lax.iota(jnp.int32, _L)
        zero = lane * 0

        def unpack_flags(b):
            for g in range(n_grp):
                w = pbits[b][pl.ds(g * _L, _L)]
                rows = lane + g * _L
                for p in range(P):
                    bit = (w >> p) & 1
                    plsc.store_scatter(pflag[b], [rows, zero + p],
                                       bit.astype(jnp.float32))

        for b in range(_NBUF):
            start_gathers(b, b)

        def outer(g, carry):
            bg = g * _NBUF
            for b in range(_NBUF):
                bloc = bg + b
                bb = b0 + bloc
                wait_tab_gathers(bloc, b)
                pltpu.async_copy(
                    trows[b].at[pl.ds(0, S)],
                    out_hbm.at[bb, :, pl.ds(P, H)], wsem[b])
                wait_bits_gathers(bloc, b)
                unpack_flags(b)
                pltpu.async_copy(
                    pflag[b].at[pl.ds(0, S)],
                    out_hbm.at[bb, :, pl.ds(0, P)], wsem_p[b])
            for b in range(_NBUF):
                bloc = bg + b
                bb = b0 + bloc
                pltpu.make_async_copy(
                    trows[b].at[pl.ds(0, S)],
                    out_hbm.at[bb, :, pl.ds(P, H)], wsem[b]).wait()
                pltpu.make_async_copy(
                    pflag[b].at[pl.ds(0, S)],
                    out_hbm.at[bb, :, pl.ds(0, P)], wsem_p[b]).wait()

                @pl.when(g < n_outer - 1)
                def _():
                    start_gathers(bloc + _NBUF, b)

            return carry

        lax.fori_loop(0, n_outer, outer, 0)

    return gather


def kernel(x, table, pos_vectors):
    B, S = x.shape
    V, H = table.shape
    P = pos_vectors.shape[0]
    weights = (2 ** jnp.arange(P, dtype=jnp.int32)).astype(jnp.float32)
    pos_bits = (weights @ pos_vectors).astype(jnp.int32)
    idx = x.reshape(B * S).astype(jnp.int32)
    out = _make_gather(B, S, V, H, P)(table, pos_bits, idx)
    return out[:, :, :P + H]
